# baseline (speedup 1.0000x reference)
import functools

import jax
import jax.numpy as jnp
from jax import lax
from jax.experimental import pallas as pl
from jax.experimental.pallas import tpu as pltpu
from jax.experimental.pallas import tpu_sc as plsc

_NC = 2
_NS = 16
_CHUNK = 128
_RB = 400


@functools.lru_cache(maxsize=None)
def _make_spmm(n_pad, n_rows, width):
    rows_per_core = n_rows // _NC
    rpt = rows_per_core // _NS
    npt = n_pad // _NS

    mesh = plsc.VectorSubcoreMesh(core_axis_name="c", subcore_axis_name="s")

    nbuf = 4
    look = 3
    assert rpt % nbuf == 0 and rpt >= nbuf

    @functools.partial(
        pl.kernel,
        out_type=jax.ShapeDtypeStruct((_NC, n_pad, width), jnp.float32),
        mesh=mesh,
        scratch_types=(
            [pltpu.VMEM((rpt, _CHUNK), jnp.int32),
             pltpu.VMEM((rpt, _CHUNK), jnp.int32)]
            + [pltpu.VMEM((_CHUNK, width), jnp.float32)] * nbuf
            + [pltpu.VMEM_SHARED((n_pad, width), jnp.float32)]
            + [pltpu.SemaphoreType.DMA] * nbuf
        ),
    )
    def spmm(table, src2d, dst2d, zeros, out, src_v, dst_v, *bufs):
        rows = bufs[:nbuf]
        acc = bufs[nbuf]
        gsem = bufs[nbuf + 1:]
        c = lax.axis_index("c")
        s = lax.axis_index("s")
        row_base = c * rows_per_core + s * rpt

        def gstart(j, b):
            pltpu.async_copy(table.at[src_v.at[j]], rows[b], gsem[b])

        def gwait(j, b):
            pltpu.make_async_copy(table.at[src_v.at[j]], rows[b], gsem[b]).wait()

        pltpu.sync_copy(zeros.at[pl.ds(s * npt, npt)], acc.at[pl.ds(s * npt, npt)])
        pltpu.sync_copy(src2d.at[pl.ds(row_base, rpt)], src_v)
        pltpu.sync_copy(dst2d.at[pl.ds(row_base, rpt)], dst_v)
        plsc.subcore_barrier()

        def body(j, carry):
            pltpu.async_copy(table.at[src_v.at[j]], rows[0], gsem[0]).wait()
            pltpu.sync_copy(rows[0], acc.at[dst_v.at[j]], add=True)
            return carry

        lax.fori_loop(0, rpt, body, 0)

        plsc.subcore_barrier()
        pltpu.sync_copy(acc.at[pl.ds(s * npt, npt)],
                        out.at[c, pl.ds(s * npt, npt)])

    return spmm


@functools.lru_cache(maxsize=None)
def _make_deg(n_pad, n_rows):
    n_workers = _NC * _NS
    rpt = n_rows // n_workers

    mesh = plsc.VectorSubcoreMesh(core_axis_name="c", subcore_axis_name="s")

    @functools.partial(
        pl.kernel,
        out_type=jax.ShapeDtypeStruct((n_workers, n_pad), jnp.float32),
        mesh=mesh,
        compiler_params=pltpu.CompilerParams(needs_layout_passes=False),
        scratch_types=[
            pltpu.VMEM((rpt, _CHUNK), jnp.int32),
            pltpu.VMEM((n_pad,), jnp.float32),
        ],
    )
    def deg(dst2d, zeros, out, dst_v, acc):
        c = lax.axis_index("c")
        s = lax.axis_index("s")
        w = s * _NC + c
        pltpu.sync_copy(zeros, acc)
        pltpu.sync_copy(dst2d.at[pl.ds(w * rpt, rpt)], dst_v)

        def body(j, carry):
            ones_v = jnp.ones((16,), jnp.float32)
            for g in range(_CHUNK // 16):
                idx = dst_v[j, pl.ds(g * 16, 16)]
                plsc.addupdate_scatter(acc, [idx], ones_v)
            return carry

        lax.fori_loop(0, rpt, body, 0)
        pltpu.sync_copy(acc, out.at[w])

    return deg


def _tc_prep(degt_ref, x_ref, dinv_ref, xt_ref):
    deg = 1.0 + jnp.sum(degt_ref[...], axis=1, keepdims=True)
    dinv = lax.rsqrt(deg)
    dinv_ref[...] = dinv
    xt_ref[...] = x_ref[...] * dinv


def _tc_l1(p_ref, xt_ref, dinv_ref, W1_ref, b1_ref, W12_ref, h2t_ref):
    dinv = dinv_ref[...]
    ax = (p_ref[0] + p_ref[1] + xt_ref[...]) * dinv
    y1 = jnp.dot(ax, W1_ref[...], preferred_element_type=jnp.float32) + b1_ref[0:1, :]
    h2 = jnp.dot(y1, W12_ref[...], preferred_element_type=jnp.float32)
    h2t_ref[...] = h2 * dinv


def _tc_l2(q_ref, h2t_ref, dinv_ref, b12_ref, y2t_ref):
    dinv = dinv_ref[...]
    y2 = (q_ref[0] + q_ref[1] + h2t_ref[...]) * dinv + b12_ref[0:1, :]
    y2t_ref[...] = jnp.maximum(y2, 0.0) * dinv


def _tc_fin(r_ref, y2t_ref, dinv_ref, W2_ref, b2_ref, out_ref):
    ay2 = (r_ref[0] + r_ref[1] + y2t_ref[...]) * dinv_ref[...]
    out_ref[...] = (jnp.dot(ay2, W2_ref[...], preferred_element_type=jnp.float32)
                    + b2_ref[0:1, :])


def _row_spec(width):
    return pl.BlockSpec((_RB, width), lambda i: (i, 0))


def _pair_spec(width):
    return pl.BlockSpec((_NC, _RB, width), lambda i: (0, i, 0))


def _full_spec(shape):
    nd = len(shape)
    return pl.BlockSpec(shape, lambda i: (0,) * nd)


def kernel(x, edge_index, W1, b1, W12, b12, W2, b2):
    n_nodes, d_in = x.shape
    n_edges = edge_index.shape[1]
    d_mid = W1.shape[1]
    d_mid2 = W12.shape[1]
    d_out = W2.shape[1]
    grid = (n_nodes // _RB,)

    edge_align = _NC * _NS * 8 * _CHUNK
    e_pad = -(-n_edges // edge_align) * edge_align
    n_pad = -(-(n_nodes + 1) // (_NS * 8)) * (_NS * 8)
    n_rows = e_pad // _CHUNK

    ei = edge_index.astype(jnp.int32)
    pad_src = jnp.zeros((e_pad - n_edges,), jnp.int32)
    pad_dst = jnp.full((e_pad - n_edges,), n_nodes, jnp.int32)
    src2d = jnp.concatenate([ei[0], pad_src]).reshape(n_rows, _CHUNK)
    dst2d = jnp.concatenate([ei[1], pad_dst]).reshape(n_rows, _CHUNK)
    z1 = jnp.zeros((n_pad,), jnp.float32)
    z128 = jnp.zeros((n_pad, d_mid2), jnp.float32)
    b1r = jnp.broadcast_to(b1[None, :], (8, d_mid))
    b12r = jnp.broadcast_to(b12[None, :], (8, d_mid2))
    b2r = jnp.broadcast_to(b2[None, :], (8, d_out))

    degp = _make_deg(n_pad, n_rows)(dst2d, z1)
    degt = degp.T

    dinv, xt = pl.pallas_call(
        _tc_prep,
        grid=grid,
        in_specs=[pl.BlockSpec((_RB, _NC * _NS), lambda i: (i, 0)),
                  _row_spec(d_in)],
        out_specs=[_row_spec(1), _row_spec(d_in)],
        out_shape=[
            jax.ShapeDtypeStruct((n_nodes, 1), jnp.float32),
            jax.ShapeDtypeStruct((n_nodes, d_in), jnp.float32),
        ],
    )(degt, x)

    p = _make_spmm(n_pad, n_rows, d_in)(xt, src2d, dst2d, z128)

    h2t = pl.pallas_call(
        _tc_l1,
        grid=grid,
        in_specs=[_pair_spec(d_in), _row_spec(d_in), _row_spec(1),
                  _full_spec(W1.shape), _full_spec((8, d_mid)),
                  _full_spec(W12.shape)],
        out_specs=_row_spec(d_mid2),
        out_shape=jax.ShapeDtypeStruct((n_nodes, d_mid2), jnp.float32),
    )(p, xt, dinv, W1, b1r, W12)

    q = _make_spmm(n_pad, n_rows, d_mid2)(h2t, src2d, dst2d, z128)

    y2t = pl.pallas_call(
        _tc_l2,
        grid=grid,
        in_specs=[_pair_spec(d_mid2), _row_spec(d_mid2), _row_spec(1),
                  _full_spec((8, d_mid2))],
        out_specs=_row_spec(d_mid2),
        out_shape=jax.ShapeDtypeStruct((n_nodes, d_mid2), jnp.float32),
    )(q, h2t, dinv, b12r)

    r = _make_spmm(n_pad, n_rows, d_mid2)(y2t, src2d, dst2d, z128)

    out = pl.pallas_call(
        _tc_fin,
        grid=grid,
        in_specs=[_pair_spec(d_mid2), _row_spec(d_mid2), _row_spec(1),
                  _full_spec(W2.shape), _full_spec((8, d_out))],
        out_specs=_row_spec(d_out),
        out_shape=jax.ShapeDtypeStruct((n_nodes, d_out), jnp.float32),
    )(r, y2t, dinv, W2, b2r)

    return out


# speedup vs baseline: 1.0038x; 1.0038x over previous
import functools

import jax
import jax.numpy as jnp
from jax import lax
from jax.experimental import pallas as pl
from jax.experimental.pallas import tpu as pltpu
from jax.experimental.pallas import tpu_sc as plsc

_NC = 2
_NS = 16
_CHUNK = 128
_RB = 400


@functools.lru_cache(maxsize=None)
def _make_spmm(n_pad, n_rows, width):
    rows_per_core = n_rows // _NC
    rpt = rows_per_core // _NS
    npt = n_pad // _NS

    mesh = plsc.VectorSubcoreMesh(core_axis_name="c", subcore_axis_name="s")

    nbuf = 4
    look = 3
    assert rpt % nbuf == 0 and rpt >= nbuf

    @functools.partial(
        pl.kernel,
        out_type=jax.ShapeDtypeStruct((_NC, n_pad, width), jnp.float32),
        mesh=mesh,
        scratch_types=(
            [pltpu.VMEM((rpt, _CHUNK), jnp.int32),
             pltpu.VMEM((rpt, _CHUNK), jnp.int32)]
            + [pltpu.VMEM((_CHUNK, width), jnp.float32)] * nbuf
            + [pltpu.VMEM_SHARED((n_pad, width), jnp.float32)]
            + [pltpu.SemaphoreType.DMA] * nbuf
        ),
    )
    def spmm(table, src2d, dst2d, zeros, out, src_v, dst_v, *bufs):
        rows = bufs[:nbuf]
        acc = bufs[nbuf]
        gsem = bufs[nbuf + 1:]
        c = lax.axis_index("c")
        s = lax.axis_index("s")
        row_base = c * rows_per_core + s * rpt

        def gstart(j, b):
            pltpu.async_copy(table.at[src_v.at[j]], rows[b], gsem[b])

        def gwait(j, b):
            pltpu.make_async_copy(table.at[src_v.at[j]], rows[b], gsem[b]).wait()

        pltpu.sync_copy(zeros.at[pl.ds(s * npt, npt)], acc.at[pl.ds(s * npt, npt)])
        pltpu.sync_copy(src2d.at[pl.ds(row_base, rpt)], src_v)
        pltpu.sync_copy(dst2d.at[pl.ds(row_base, rpt)], dst_v)
        plsc.subcore_barrier()

        def body(j, carry):
            pltpu.async_copy(table.at[src_v.at[j]], rows[0], gsem[0]).wait()
            pltpu.sync_copy(rows[0], acc.at[dst_v.at[j]], add=True)
            return carry

        lax.fori_loop(0, rpt, body, 0)

        plsc.subcore_barrier()
        pltpu.sync_copy(acc.at[pl.ds(s * npt, npt)],
                        out.at[c, pl.ds(s * npt, npt)])

    return spmm


@functools.lru_cache(maxsize=None)
def _make_deg(n_pad, n_rows):
    n_workers = _NC * _NS
    rpt = n_rows // n_workers

    mesh = plsc.VectorSubcoreMesh(core_axis_name="c", subcore_axis_name="s")

    @functools.partial(
        pl.kernel,
        out_type=jax.ShapeDtypeStruct((n_workers, n_pad), jnp.float32),
        mesh=mesh,
        compiler_params=pltpu.CompilerParams(needs_layout_passes=False),
        scratch_types=[
            pltpu.VMEM((rpt, _CHUNK), jnp.int32),
            pltpu.VMEM((n_pad,), jnp.float32),
        ],
    )
    def deg(dst2d, zeros, out, dst_v, acc):
        c = lax.axis_index("c")
        s = lax.axis_index("s")
        w = s * _NC + c
        pltpu.sync_copy(zeros, acc)
        pltpu.sync_copy(dst2d.at[pl.ds(w * rpt, rpt)], dst_v)

        def body(j, carry):
            ones_v = jnp.ones((16,), jnp.float32)
            for g in range(_CHUNK // 16):
                idx = dst_v[j, pl.ds(g * 16, 16)]
                plsc.addupdate_scatter(acc, [idx], ones_v)
            return carry

        lax.fori_loop(0, rpt, body, 0)
        pltpu.sync_copy(acc, out.at[w])

    return deg


def _tc_prep(degt_ref, x_ref, dinv_ref, xt_ref):
    deg = 1.0 + jnp.sum(degt_ref[...], axis=1, keepdims=True)
    dinv = lax.rsqrt(deg)
    dinv_ref[...] = dinv
    xt_ref[...] = x_ref[...] * dinv


def _tc_l1(p_ref, xt_ref, dinv_ref, W1_ref, b1_ref, W12_ref, h2t_ref):
    dinv = dinv_ref[...]
    ax = (p_ref[0] + p_ref[1] + xt_ref[...]) * dinv
    y1 = jnp.dot(ax, W1_ref[...], preferred_element_type=jnp.float32) + b1_ref[0:1, :]
    h2 = jnp.dot(y1, W12_ref[...], preferred_element_type=jnp.float32)
    h2t_ref[...] = h2 * dinv


def _tc_l2(q_ref, h2t_ref, dinv_ref, b12_ref, y2t_ref):
    dinv = dinv_ref[...]
    y2 = (q_ref[0] + q_ref[1] + h2t_ref[...]) * dinv + b12_ref[0:1, :]
    y2t_ref[...] = jnp.maximum(y2, 0.0) * dinv


def _tc_fin(r_ref, y2t_ref, dinv_ref, W2_ref, b2_ref, out_ref):
    ay2 = (r_ref[0] + r_ref[1] + y2t_ref[...]) * dinv_ref[...]
    out_ref[...] = (jnp.dot(ay2, W2_ref[...], preferred_element_type=jnp.float32)
                    + b2_ref[0:1, :])


def _row_spec(width):
    return pl.BlockSpec((_RB, width), lambda i: (i, 0))


def _pair_spec(width):
    return pl.BlockSpec((_NC, _RB, width), lambda i: (0, i, 0))


def _full_spec(shape):
    nd = len(shape)
    return pl.BlockSpec(shape, lambda i: (0,) * nd)


def kernel(x, edge_index, W1, b1, W12, b12, W2, b2):
    n_nodes, d_in = x.shape
    n_edges = edge_index.shape[1]
    d_mid = W1.shape[1]
    d_mid2 = W12.shape[1]
    d_out = W2.shape[1]
    grid = (n_nodes // _RB,)

    edge_align = _NC * _NS * 8 * _CHUNK
    e_pad = -(-n_edges // edge_align) * edge_align
    n_pad = -(-(n_nodes + 1) // (_NS * 8)) * (_NS * 8)
    n_rows = e_pad // _CHUNK

    ei = edge_index.astype(jnp.int32)
    pad_src = jnp.zeros((e_pad - n_edges,), jnp.int32)
    n_trash = n_pad - n_nodes
    pad_dst = n_nodes + jnp.arange(e_pad - n_edges, dtype=jnp.int32) % n_trash
    src2d = jnp.concatenate([ei[0], pad_src]).reshape(n_rows, _CHUNK)
    dst2d = jnp.concatenate([ei[1], pad_dst]).reshape(n_rows, _CHUNK)
    z1 = jnp.zeros((n_pad,), jnp.float32)
    z128 = jnp.zeros((n_pad, d_mid2), jnp.float32)
    b1r = jnp.broadcast_to(b1[None, :], (8, d_mid))
    b12r = jnp.broadcast_to(b12[None, :], (8, d_mid2))
    b2r = jnp.broadcast_to(b2[None, :], (8, d_out))

    degp = _make_deg(n_pad, n_rows)(dst2d, z1)
    degt = degp.T

    dinv, xt = pl.pallas_call(
        _tc_prep,
        grid=grid,
        in_specs=[pl.BlockSpec((_RB, _NC * _NS), lambda i: (i, 0)),
                  _row_spec(d_in)],
        out_specs=[_row_spec(1), _row_spec(d_in)],
        out_shape=[
            jax.ShapeDtypeStruct((n_nodes, 1), jnp.float32),
            jax.ShapeDtypeStruct((n_nodes, d_in), jnp.float32),
        ],
    )(degt, x)

    p = _make_spmm(n_pad, n_rows, d_in)(xt, src2d, dst2d, z128)

    h2t = pl.pallas_call(
        _tc_l1,
        grid=grid,
        in_specs=[_pair_spec(d_in), _row_spec(d_in), _row_spec(1),
                  _full_spec(W1.shape), _full_spec((8, d_mid)),
                  _full_spec(W12.shape)],
        out_specs=_row_spec(d_mid2),
        out_shape=jax.ShapeDtypeStruct((n_nodes, d_mid2), jnp.float32),
    )(p, xt, dinv, W1, b1r, W12)

    q = _make_spmm(n_pad, n_rows, d_mid2)(h2t, src2d, dst2d, z128)

    y2t = pl.pallas_call(
        _tc_l2,
        grid=grid,
        in_specs=[_pair_spec(d_mid2), _row_spec(d_mid2), _row_spec(1),
                  _full_spec((8, d_mid2))],
        out_specs=_row_spec(d_mid2),
        out_shape=jax.ShapeDtypeStruct((n_nodes, d_mid2), jnp.float32),
    )(q, h2t, dinv, b12r)

    r = _make_spmm(n_pad, n_rows, d_mid2)(y2t, src2d, dst2d, z128)

    out = pl.pallas_call(
        _tc_fin,
        grid=grid,
        in_specs=[_pair_spec(d_mid2), _row_spec(d_mid2), _row_spec(1),
                  _full_spec(W2.shape), _full_spec((8, d_out))],
        out_specs=_row_spec(d_out),
        out_shape=jax.ShapeDtypeStruct((n_nodes, d_out), jnp.float32),
    )(r, y2t, dinv, W2, b2r)

    return out


# speedup vs baseline: 1.1551x; 1.1507x over previous
import functools

import jax
import jax.numpy as jnp
from jax import lax
from jax.experimental import pallas as pl
from jax.experimental.pallas import tpu as pltpu
from jax.experimental.pallas import tpu_sc as plsc

_NC = 2
_NS = 16
_CHUNK = 128
_RB = 400


@functools.lru_cache(maxsize=None)
def _make_spmm(n_pad, n_rows, width, rpt0_frac=0.75):
    rpt_pair = n_rows // _NS
    rpt0 = int(round(rpt_pair * rpt0_frac / 8)) * 8
    rpt1 = rpt_pair - rpt0
    npt = n_pad // _NS

    mesh = plsc.VectorSubcoreMesh(core_axis_name="c", subcore_axis_name="s")

    @functools.partial(
        pl.kernel,
        out_type=jax.ShapeDtypeStruct((_NC, n_pad, width), jnp.float32),
        mesh=mesh,
        scratch_types=[
            pltpu.VMEM((rpt0, _CHUNK), jnp.int32),
            pltpu.VMEM((rpt0, _CHUNK), jnp.int32),
            pltpu.VMEM((_CHUNK, width), jnp.float32),
            pltpu.VMEM_SHARED((n_pad, width), jnp.float32),
            pltpu.SemaphoreType.DMA,
        ],
    )
    def spmm(table, src2d, dst2d, zeros, out, src_v, dst_v, rows, acc, gsem):
        c = lax.axis_index("c")
        s = lax.axis_index("s")
        pltpu.sync_copy(zeros.at[pl.ds(s * npt, npt)], acc.at[pl.ds(s * npt, npt)])
        plsc.subcore_barrier()

        def run_share(base, n):
            pltpu.sync_copy(src2d.at[pl.ds(base, n)], src_v.at[pl.ds(0, n)])
            pltpu.sync_copy(dst2d.at[pl.ds(base, n)], dst_v.at[pl.ds(0, n)])

            def body(j, carry):
                pltpu.async_copy(table.at[src_v.at[j]], rows, gsem).wait()
                pltpu.sync_copy(rows, acc.at[dst_v.at[j]], add=True)
                return carry

            lax.fori_loop(0, n, body, 0)

        @pl.when(c == 0)
        def _():
            run_share(s * rpt0, rpt0)

        @pl.when(c == 1)
        def _():
            run_share(_NS * rpt0 + s * rpt1, rpt1)

        plsc.subcore_barrier()
        pltpu.sync_copy(acc.at[pl.ds(s * npt, npt)],
                        out.at[c, pl.ds(s * npt, npt)])

    return spmm


@functools.lru_cache(maxsize=None)
def _make_deg(n_pad, n_rows):
    n_workers = _NC * _NS
    rpt = n_rows // n_workers

    mesh = plsc.VectorSubcoreMesh(core_axis_name="c", subcore_axis_name="s")

    @functools.partial(
        pl.kernel,
        out_type=jax.ShapeDtypeStruct((n_workers, n_pad), jnp.float32),
        mesh=mesh,
        compiler_params=pltpu.CompilerParams(needs_layout_passes=False),
        scratch_types=[
            pltpu.VMEM((rpt, _CHUNK), jnp.int32),
            pltpu.VMEM((n_pad,), jnp.float32),
        ],
    )
    def deg(dst2d, zeros, out, dst_v, acc):
        c = lax.axis_index("c")
        s = lax.axis_index("s")
        w = s * _NC + c
        pltpu.sync_copy(zeros, acc)
        pltpu.sync_copy(dst2d.at[pl.ds(w * rpt, rpt)], dst_v)

        def body(j, carry):
            ones_v = jnp.ones((16,), jnp.float32)
            for g in range(_CHUNK // 16):
                idx = dst_v[j, pl.ds(g * 16, 16)]
                plsc.addupdate_scatter(acc, [idx], ones_v)
            return carry

        lax.fori_loop(0, rpt, body, 0)
        pltpu.sync_copy(acc, out.at[w])

    return deg


def _tc_prep(degt_ref, x_ref, dinv_ref, xt_ref):
    deg = 1.0 + jnp.sum(degt_ref[...], axis=1, keepdims=True)
    dinv = lax.rsqrt(deg)
    dinv_ref[...] = dinv
    xt_ref[...] = x_ref[...] * dinv


def _tc_l1(p_ref, xt_ref, dinv_ref, W1_ref, b1_ref, W12_ref, h2t_ref):
    dinv = dinv_ref[...]
    ax = (p_ref[0] + p_ref[1] + xt_ref[...]) * dinv
    y1 = jnp.dot(ax, W1_ref[...], preferred_element_type=jnp.float32) + b1_ref[0:1, :]
    h2 = jnp.dot(y1, W12_ref[...], preferred_element_type=jnp.float32)
    h2t_ref[...] = h2 * dinv


def _tc_l2(q_ref, h2t_ref, dinv_ref, b12_ref, y2t_ref):
    dinv = dinv_ref[...]
    y2 = (q_ref[0] + q_ref[1] + h2t_ref[...]) * dinv + b12_ref[0:1, :]
    y2t_ref[...] = jnp.maximum(y2, 0.0) * dinv


def _tc_fin(r_ref, y2t_ref, dinv_ref, W2_ref, b2_ref, out_ref):
    ay2 = (r_ref[0] + r_ref[1] + y2t_ref[...]) * dinv_ref[...]
    out_ref[...] = (jnp.dot(ay2, W2_ref[...], preferred_element_type=jnp.float32)
                    + b2_ref[0:1, :])


def _row_spec(width):
    return pl.BlockSpec((_RB, width), lambda i: (i, 0))


def _pair_spec(width):
    return pl.BlockSpec((_NC, _RB, width), lambda i: (0, i, 0))


def _full_spec(shape):
    nd = len(shape)
    return pl.BlockSpec(shape, lambda i: (0,) * nd)


def kernel(x, edge_index, W1, b1, W12, b12, W2, b2):
    n_nodes, d_in = x.shape
    n_edges = edge_index.shape[1]
    d_mid = W1.shape[1]
    d_mid2 = W12.shape[1]
    d_out = W2.shape[1]
    grid = (n_nodes // _RB,)

    edge_align = _NC * _NS * 8 * _CHUNK
    e_pad = -(-n_edges // edge_align) * edge_align
    n_pad = -(-(n_nodes + 1) // (_NS * 8)) * (_NS * 8)
    n_rows = e_pad // _CHUNK

    ei = edge_index.astype(jnp.int32)
    pad_src = jnp.zeros((e_pad - n_edges,), jnp.int32)
    n_trash = n_pad - n_nodes
    pad_dst = n_nodes + jnp.arange(e_pad - n_edges, dtype=jnp.int32) % n_trash
    src2d = jnp.concatenate([ei[0], pad_src]).reshape(n_rows, _CHUNK)
    dst2d = jnp.concatenate([ei[1], pad_dst]).reshape(n_rows, _CHUNK)
    z1 = jnp.zeros((n_pad,), jnp.float32)
    z128 = jnp.zeros((n_pad, d_mid2), jnp.float32)
    b1r = jnp.broadcast_to(b1[None, :], (8, d_mid))
    b12r = jnp.broadcast_to(b12[None, :], (8, d_mid2))
    b2r = jnp.broadcast_to(b2[None, :], (8, d_out))

    degp = _make_deg(n_pad, n_rows)(dst2d, z1)
    degt = degp.T

    dinv, xt = pl.pallas_call(
        _tc_prep,
        grid=grid,
        in_specs=[pl.BlockSpec((_RB, _NC * _NS), lambda i: (i, 0)),
                  _row_spec(d_in)],
        out_specs=[_row_spec(1), _row_spec(d_in)],
        out_shape=[
            jax.ShapeDtypeStruct((n_nodes, 1), jnp.float32),
            jax.ShapeDtypeStruct((n_nodes, d_in), jnp.float32),
        ],
    )(degt, x)

    p = _make_spmm(n_pad, n_rows, d_in)(xt, src2d, dst2d, z128)

    h2t = pl.pallas_call(
        _tc_l1,
        grid=grid,
        in_specs=[_pair_spec(d_in), _row_spec(d_in), _row_spec(1),
                  _full_spec(W1.shape), _full_spec((8, d_mid)),
                  _full_spec(W12.shape)],
        out_specs=_row_spec(d_mid2),
        out_shape=jax.ShapeDtypeStruct((n_nodes, d_mid2), jnp.float32),
    )(p, xt, dinv, W1, b1r, W12)

    q = _make_spmm(n_pad, n_rows, d_mid2)(h2t, src2d, dst2d, z128)

    y2t = pl.pallas_call(
        _tc_l2,
        grid=grid,
        in_specs=[_pair_spec(d_mid2), _row_spec(d_mid2), _row_spec(1),
                  _full_spec((8, d_mid2))],
        out_specs=_row_spec(d_mid2),
        out_shape=jax.ShapeDtypeStruct((n_nodes, d_mid2), jnp.float32),
    )(q, h2t, dinv, b12r)

    r = _make_spmm(n_pad, n_rows, d_mid2)(y2t, src2d, dst2d, z128)

    out = pl.pallas_call(
        _tc_fin,
        grid=grid,
        in_specs=[_pair_spec(d_mid2), _row_spec(d_mid2), _row_spec(1),
                  _full_spec(W2.shape), _full_spec((8, d_out))],
        out_specs=_row_spec(d_out),
        out_shape=jax.ShapeDtypeStruct((n_nodes, d_out), jnp.float32),
    )(r, y2t, dinv, W2, b2r)

    return out


# speedup vs baseline: 1.1858x; 1.0266x over previous
import functools

import jax
import jax.numpy as jnp
from jax import lax
from jax.experimental import pallas as pl
from jax.experimental.pallas import tpu as pltpu
from jax.experimental.pallas import tpu_sc as plsc

_NC = 2
_NS = 16
_CHUNK = 128
_RB = 400


@functools.lru_cache(maxsize=None)
def _make_spmm(n_pad, n_rows, width, rpt0_frac=0.75):
    rpt_pair = n_rows // _NS
    rpt0 = int(round(rpt_pair * rpt0_frac / 8)) * 8
    rpt1 = rpt_pair - rpt0
    npt = n_pad // _NS

    mesh = plsc.VectorSubcoreMesh(core_axis_name="c", subcore_axis_name="s")

    @functools.partial(
        pl.kernel,
        out_type=jax.ShapeDtypeStruct((_NC, n_pad, width), jnp.float32),
        mesh=mesh,
        scratch_types=[
            pltpu.VMEM((rpt0, _CHUNK), jnp.int32),
            pltpu.VMEM((rpt0, _CHUNK), jnp.int32),
            pltpu.VMEM((_CHUNK, width), jnp.float32),
            pltpu.VMEM_SHARED((n_pad, width), jnp.float32),
            pltpu.SemaphoreType.DMA,
        ],
    )
    def spmm(table, src2d, dst2d, out, src_v, dst_v, rows, acc, gsem):
        c = lax.axis_index("c")
        s = lax.axis_index("s")

        def zfill(i, carry):
            for g in range(width // 16):
                rows[i, pl.ds(g * 16, 16)] = jnp.zeros((16,), jnp.float32)
            return carry

        lax.fori_loop(0, _CHUNK, zfill, 0)
        nfull = npt // _CHUNK
        rem = npt % _CHUNK
        for k in range(nfull):
            pltpu.sync_copy(rows, acc.at[pl.ds(s * npt + k * _CHUNK, _CHUNK)])
        if rem:
            pltpu.sync_copy(rows.at[pl.ds(0, rem)],
                            acc.at[pl.ds(s * npt + nfull * _CHUNK, rem)])
        plsc.subcore_barrier()

        def run_share(base, n):
            pltpu.sync_copy(src2d.at[pl.ds(base, n)], src_v.at[pl.ds(0, n)])
            pltpu.sync_copy(dst2d.at[pl.ds(base, n)], dst_v.at[pl.ds(0, n)])

            def body(j, carry):
                pltpu.async_copy(table.at[src_v.at[j]], rows, gsem).wait()
                pltpu.sync_copy(rows, acc.at[dst_v.at[j]], add=True)
                return carry

            lax.fori_loop(0, n, body, 0)

        @pl.when(c == 0)
        def _():
            run_share(s * rpt0, rpt0)

        @pl.when(c == 1)
        def _():
            run_share(_NS * rpt0 + s * rpt1, rpt1)

        plsc.subcore_barrier()
        for k in range(nfull):
            off = s * npt + k * _CHUNK
            pltpu.sync_copy(acc.at[pl.ds(off, _CHUNK)], rows)
            pltpu.sync_copy(rows, out.at[c, pl.ds(off, _CHUNK)])
        if rem:
            off = s * npt + nfull * _CHUNK
            pltpu.sync_copy(acc.at[pl.ds(off, rem)], rows.at[pl.ds(0, rem)])
            pltpu.sync_copy(rows.at[pl.ds(0, rem)], out.at[c, pl.ds(off, rem)])

    return spmm


@functools.lru_cache(maxsize=None)
def _make_deg(n_pad, n_rows):
    n_workers = _NC * _NS
    rpt = n_rows // n_workers

    mesh = plsc.VectorSubcoreMesh(core_axis_name="c", subcore_axis_name="s")

    @functools.partial(
        pl.kernel,
        out_type=jax.ShapeDtypeStruct((n_workers, n_pad), jnp.float32),
        mesh=mesh,
        compiler_params=pltpu.CompilerParams(needs_layout_passes=False),
        scratch_types=[
            pltpu.VMEM((rpt, _CHUNK), jnp.int32),
            pltpu.VMEM((n_pad,), jnp.float32),
        ],
    )
    def deg(dst2d, zeros, out, dst_v, acc):
        c = lax.axis_index("c")
        s = lax.axis_index("s")
        w = s * _NC + c
        pltpu.sync_copy(zeros, acc)
        pltpu.sync_copy(dst2d.at[pl.ds(w * rpt, rpt)], dst_v)

        def body(j, carry):
            ones_v = jnp.ones((16,), jnp.float32)
            for g in range(_CHUNK // 16):
                idx = dst_v[j, pl.ds(g * 16, 16)]
                plsc.addupdate_scatter(acc, [idx], ones_v)
            return carry

        lax.fori_loop(0, rpt, body, 0)
        pltpu.sync_copy(acc, out.at[w])

    return deg


def _tc_prep(degt_ref, x_ref, dinv_ref, xt_ref):
    deg = 1.0 + jnp.sum(degt_ref[...], axis=1, keepdims=True)
    dinv = lax.rsqrt(deg)
    dinv_ref[...] = dinv
    xt_ref[...] = x_ref[...] * dinv


def _tc_l1(p_ref, xt_ref, dinv_ref, W1_ref, b1_ref, W12_ref, h2t_ref):
    dinv = dinv_ref[...]
    ax = (p_ref[0] + p_ref[1] + xt_ref[...]) * dinv
    y1 = jnp.dot(ax, W1_ref[...], preferred_element_type=jnp.float32) + b1_ref[0:1, :]
    h2 = jnp.dot(y1, W12_ref[...], preferred_element_type=jnp.float32)
    h2t_ref[...] = h2 * dinv


def _tc_l2(q_ref, h2t_ref, dinv_ref, b12_ref, y2t_ref):
    dinv = dinv_ref[...]
    y2 = (q_ref[0] + q_ref[1] + h2t_ref[...]) * dinv + b12_ref[0:1, :]
    y2t_ref[...] = jnp.maximum(y2, 0.0) * dinv


def _tc_fin(r_ref, y2t_ref, dinv_ref, W2_ref, b2_ref, out_ref):
    ay2 = (r_ref[0] + r_ref[1] + y2t_ref[...]) * dinv_ref[...]
    out_ref[...] = (jnp.dot(ay2, W2_ref[...], preferred_element_type=jnp.float32)
                    + b2_ref[0:1, :])


def _row_spec(width):
    return pl.BlockSpec((_RB, width), lambda i: (i, 0))


def _pair_spec(width):
    return pl.BlockSpec((_NC, _RB, width), lambda i: (0, i, 0))


def _full_spec(shape):
    nd = len(shape)
    return pl.BlockSpec(shape, lambda i: (0,) * nd)


def kernel(x, edge_index, W1, b1, W12, b12, W2, b2):
    n_nodes, d_in = x.shape
    n_edges = edge_index.shape[1]
    d_mid = W1.shape[1]
    d_mid2 = W12.shape[1]
    d_out = W2.shape[1]
    grid = (n_nodes // _RB,)

    edge_align = _NC * _NS * 8 * _CHUNK
    e_pad = -(-n_edges // edge_align) * edge_align
    n_pad = -(-(n_nodes + 1) // (_NS * 8)) * (_NS * 8)
    n_rows = e_pad // _CHUNK

    ei = edge_index.astype(jnp.int32)
    pad_src = jnp.zeros((e_pad - n_edges,), jnp.int32)
    n_trash = n_pad - n_nodes
    pad_dst = n_nodes + jnp.arange(e_pad - n_edges, dtype=jnp.int32) % n_trash
    src2d = jnp.concatenate([ei[0], pad_src]).reshape(n_rows, _CHUNK)
    dst2d = jnp.concatenate([ei[1], pad_dst]).reshape(n_rows, _CHUNK)
    z1 = jnp.zeros((n_pad,), jnp.float32)
    b1r = jnp.broadcast_to(b1[None, :], (8, d_mid))
    b12r = jnp.broadcast_to(b12[None, :], (8, d_mid2))
    b2r = jnp.broadcast_to(b2[None, :], (8, d_out))

    degp = _make_deg(n_pad, n_rows)(dst2d, z1)
    degt = degp.T

    dinv, xt = pl.pallas_call(
        _tc_prep,
        grid=grid,
        in_specs=[pl.BlockSpec((_RB, _NC * _NS), lambda i: (i, 0)),
                  _row_spec(d_in)],
        out_specs=[_row_spec(1), _row_spec(d_in)],
        out_shape=[
            jax.ShapeDtypeStruct((n_nodes, 1), jnp.float32),
            jax.ShapeDtypeStruct((n_nodes, d_in), jnp.float32),
        ],
    )(degt, x)

    p = _make_spmm(n_pad, n_rows, d_in)(xt, src2d, dst2d)

    h2t = pl.pallas_call(
        _tc_l1,
        grid=grid,
        in_specs=[_pair_spec(d_in), _row_spec(d_in), _row_spec(1),
                  _full_spec(W1.shape), _full_spec((8, d_mid)),
                  _full_spec(W12.shape)],
        out_specs=_row_spec(d_mid2),
        out_shape=jax.ShapeDtypeStruct((n_nodes, d_mid2), jnp.float32),
    )(p, xt, dinv, W1, b1r, W12)

    q = _make_spmm(n_pad, n_rows, d_mid2)(h2t, src2d, dst2d)

    y2t = pl.pallas_call(
        _tc_l2,
        grid=grid,
        in_specs=[_pair_spec(d_mid2), _row_spec(d_mid2), _row_spec(1),
                  _full_spec((8, d_mid2))],
        out_specs=_row_spec(d_mid2),
        out_shape=jax.ShapeDtypeStruct((n_nodes, d_mid2), jnp.float32),
    )(q, h2t, dinv, b12r)

    r = _make_spmm(n_pad, n_rows, d_mid2)(y2t, src2d, dst2d)

    out = pl.pallas_call(
        _tc_fin,
        grid=grid,
        in_specs=[_pair_spec(d_mid2), _row_spec(d_mid2), _row_spec(1),
                  _full_spec(W2.shape), _full_spec((8, d_out))],
        out_specs=_row_spec(d_out),
        out_shape=jax.ShapeDtypeStruct((n_nodes, d_out), jnp.float32),
    )(r, y2t, dinv, W2, b2r)

    return out


# speedup vs baseline: 1.1908x; 1.0042x over previous
import functools

import jax
import jax.numpy as jnp
from jax import lax
from jax.experimental import pallas as pl
from jax.experimental.pallas import tpu as pltpu
from jax.experimental.pallas import tpu_sc as plsc

_NC = 2
_NS = 16
_CHUNK = 128
_RB = 400


@functools.lru_cache(maxsize=None)
def _make_spmm(n_pad, n_rows, width, rpt0_frac=0.75):
    blk = 8
    rpt_pair = n_rows // _NS
    rpt0 = int(round(rpt_pair * rpt0_frac / blk)) * blk
    rpt1 = rpt_pair - rpt0
    npt = n_pad // _NS

    mesh = plsc.VectorSubcoreMesh(core_axis_name="c", subcore_axis_name="s")

    @functools.partial(
        pl.kernel,
        out_type=jax.ShapeDtypeStruct((_NC, n_pad, width), jnp.float32),
        mesh=mesh,
        scratch_types=[
            pltpu.VMEM((blk, _CHUNK), jnp.int32),
            pltpu.VMEM((blk, _CHUNK), jnp.int32),
            pltpu.VMEM((_CHUNK, width), jnp.float32),
            pltpu.VMEM((_CHUNK, width), jnp.float32),
            pltpu.VMEM_SHARED((n_pad, width), jnp.float32),
            pltpu.SemaphoreType.DMA,
            pltpu.SemaphoreType.DMA,
        ],
    )
    def spmm(table, src2d, dst2d, out, src_v, dst_v, r0, r1, acc, g0, g1):
        rows = (r0, r1)
        gsem = (g0, g1)
        c = lax.axis_index("c")
        s = lax.axis_index("s")

        def zfill(i, carry):
            for g in range(width // 16):
                r0[i, pl.ds(g * 16, 16)] = jnp.zeros((16,), jnp.float32)
            return carry

        lax.fori_loop(0, _CHUNK, zfill, 0)
        nfull = npt // _CHUNK
        rem = npt % _CHUNK
        for k in range(nfull):
            pltpu.sync_copy(r0, acc.at[pl.ds(s * npt + k * _CHUNK, _CHUNK)])
        if rem:
            pltpu.sync_copy(r0.at[pl.ds(0, rem)],
                            acc.at[pl.ds(s * npt + nfull * _CHUNK, rem)])
        plsc.subcore_barrier()

        def run_share(base, n):
            def block(ib, carry):
                pltpu.sync_copy(src2d.at[pl.ds(base + ib * blk, blk)], src_v)
                pltpu.sync_copy(dst2d.at[pl.ds(base + ib * blk, blk)], dst_v)
                pltpu.async_copy(table.at[src_v.at[0]], rows[0], gsem[0])
                for j in range(blk):
                    b = j % 2
                    if j + 1 < blk:
                        pltpu.async_copy(table.at[src_v.at[j + 1]],
                                         rows[1 - b], gsem[1 - b])
                    pltpu.make_async_copy(table.at[src_v.at[j]], rows[b],
                                          gsem[b]).wait()
                    pltpu.sync_copy(rows[b], acc.at[dst_v.at[j]], add=True)
                return carry

            lax.fori_loop(0, n // blk, block, 0)

        @pl.when(c == 0)
        def _():
            run_share(s * rpt0, rpt0)

        @pl.when(c == 1)
        def _():
            run_share(_NS * rpt0 + s * rpt1, rpt1)

        plsc.subcore_barrier()
        for k in range(nfull):
            off = s * npt + k * _CHUNK
            pltpu.sync_copy(acc.at[pl.ds(off, _CHUNK)], r0)
            pltpu.sync_copy(r0, out.at[c, pl.ds(off, _CHUNK)])
        if rem:
            off = s * npt + nfull * _CHUNK
            pltpu.sync_copy(acc.at[pl.ds(off, rem)], r0.at[pl.ds(0, rem)])
            pltpu.sync_copy(r0.at[pl.ds(0, rem)], out.at[c, pl.ds(off, rem)])

    return spmm


@functools.lru_cache(maxsize=None)
def _make_deg(n_pad, n_rows):
    n_workers = _NC * _NS
    rpt = n_rows // n_workers

    mesh = plsc.VectorSubcoreMesh(core_axis_name="c", subcore_axis_name="s")

    @functools.partial(
        pl.kernel,
        out_type=jax.ShapeDtypeStruct((n_workers, n_pad), jnp.float32),
        mesh=mesh,
        compiler_params=pltpu.CompilerParams(needs_layout_passes=False),
        scratch_types=[
            pltpu.VMEM((rpt, _CHUNK), jnp.int32),
            pltpu.VMEM((n_pad,), jnp.float32),
        ],
    )
    def deg(dst2d, zeros, out, dst_v, acc):
        c = lax.axis_index("c")
        s = lax.axis_index("s")
        w = s * _NC + c
        pltpu.sync_copy(zeros, acc)
        pltpu.sync_copy(dst2d.at[pl.ds(w * rpt, rpt)], dst_v)

        def body(j, carry):
            ones_v = jnp.ones((16,), jnp.float32)
            for g in range(_CHUNK // 16):
                idx = dst_v[j, pl.ds(g * 16, 16)]
                plsc.addupdate_scatter(acc, [idx], ones_v)
            return carry

        lax.fori_loop(0, rpt, body, 0)
        pltpu.sync_copy(acc, out.at[w])

    return deg


def _tc_prep(degt_ref, x_ref, dinv_ref, xt_ref):
    deg = 1.0 + jnp.sum(degt_ref[...], axis=1, keepdims=True)
    dinv = lax.rsqrt(deg)
    dinv_ref[...] = dinv
    xt_ref[...] = x_ref[...] * dinv


def _tc_l1(p_ref, xt_ref, dinv_ref, W1_ref, b1_ref, W12_ref, h2t_ref):
    dinv = dinv_ref[...]
    ax = (p_ref[0] + p_ref[1] + xt_ref[...]) * dinv
    y1 = jnp.dot(ax, W1_ref[...], preferred_element_type=jnp.float32) + b1_ref[0:1, :]
    h2 = jnp.dot(y1, W12_ref[...], preferred_element_type=jnp.float32)
    h2t_ref[...] = h2 * dinv


def _tc_l2(q_ref, h2t_ref, dinv_ref, b12_ref, y2t_ref):
    dinv = dinv_ref[...]
    y2 = (q_ref[0] + q_ref[1] + h2t_ref[...]) * dinv + b12_ref[0:1, :]
    y2t_ref[...] = jnp.maximum(y2, 0.0) * dinv


def _tc_fin(r_ref, y2t_ref, dinv_ref, W2_ref, b2_ref, out_ref):
    ay2 = (r_ref[0] + r_ref[1] + y2t_ref[...]) * dinv_ref[...]
    out_ref[...] = (jnp.dot(ay2, W2_ref[...], preferred_element_type=jnp.float32)
                    + b2_ref[0:1, :])


def _row_spec(width):
    return pl.BlockSpec((_RB, width), lambda i: (i, 0))


def _pair_spec(width):
    return pl.BlockSpec((_NC, _RB, width), lambda i: (0, i, 0))


def _full_spec(shape):
    nd = len(shape)
    return pl.BlockSpec(shape, lambda i: (0,) * nd)


def kernel(x, edge_index, W1, b1, W12, b12, W2, b2):
    n_nodes, d_in = x.shape
    n_edges = edge_index.shape[1]
    d_mid = W1.shape[1]
    d_mid2 = W12.shape[1]
    d_out = W2.shape[1]
    grid = (n_nodes // _RB,)

    edge_align = _NC * _NS * 8 * _CHUNK
    e_pad = -(-n_edges // edge_align) * edge_align
    n_pad = -(-(n_nodes + 1) // (_NS * 8)) * (_NS * 8)
    n_rows = e_pad // _CHUNK

    ei = edge_index.astype(jnp.int32)
    pad_src = jnp.zeros((e_pad - n_edges,), jnp.int32)
    n_trash = n_pad - n_nodes
    pad_dst = n_nodes + jnp.arange(e_pad - n_edges, dtype=jnp.int32) % n_trash
    src2d = jnp.concatenate([ei[0], pad_src]).reshape(n_rows, _CHUNK)
    dst2d = jnp.concatenate([ei[1], pad_dst]).reshape(n_rows, _CHUNK)
    z1 = jnp.zeros((n_pad,), jnp.float32)
    b1r = jnp.broadcast_to(b1[None, :], (8, d_mid))
    b12r = jnp.broadcast_to(b12[None, :], (8, d_mid2))
    b2r = jnp.broadcast_to(b2[None, :], (8, d_out))

    degp = _make_deg(n_pad, n_rows)(dst2d, z1)
    degt = degp.T

    dinv, xt = pl.pallas_call(
        _tc_prep,
        grid=grid,
        in_specs=[pl.BlockSpec((_RB, _NC * _NS), lambda i: (i, 0)),
                  _row_spec(d_in)],
        out_specs=[_row_spec(1), _row_spec(d_in)],
        out_shape=[
            jax.ShapeDtypeStruct((n_nodes, 1), jnp.float32),
            jax.ShapeDtypeStruct((n_nodes, d_in), jnp.float32),
        ],
    )(degt, x)

    p = _make_spmm(n_pad, n_rows, d_in)(xt, src2d, dst2d)

    h2t = pl.pallas_call(
        _tc_l1,
        grid=grid,
        in_specs=[_pair_spec(d_in), _row_spec(d_in), _row_spec(1),
                  _full_spec(W1.shape), _full_spec((8, d_mid)),
                  _full_spec(W12.shape)],
        out_specs=_row_spec(d_mid2),
        out_shape=jax.ShapeDtypeStruct((n_nodes, d_mid2), jnp.float32),
    )(p, xt, dinv, W1, b1r, W12)

    q = _make_spmm(n_pad, n_rows, d_mid2)(h2t, src2d, dst2d)

    y2t = pl.pallas_call(
        _tc_l2,
        grid=grid,
        in_specs=[_pair_spec(d_mid2), _row_spec(d_mid2), _row_spec(1),
                  _full_spec((8, d_mid2))],
        out_specs=_row_spec(d_mid2),
        out_shape=jax.ShapeDtypeStruct((n_nodes, d_mid2), jnp.float32),
    )(q, h2t, dinv, b12r)

    r = _make_spmm(n_pad, n_rows, d_mid2)(y2t, src2d, dst2d)

    out = pl.pallas_call(
        _tc_fin,
        grid=grid,
        in_specs=[_pair_spec(d_mid2), _row_spec(d_mid2), _row_spec(1),
                  _full_spec(W2.shape), _full_spec((8, d_out))],
        out_specs=_row_spec(d_out),
        out_shape=jax.ShapeDtypeStruct((n_nodes, d_out), jnp.float32),
    )(r, y2t, dinv, W2, b2r)

    return out


# speedup vs baseline: 1.3936x; 1.1703x over previous
import functools

import jax
import jax.numpy as jnp
from jax import lax
from jax.experimental import pallas as pl
from jax.experimental.pallas import tpu as pltpu
from jax.experimental.pallas import tpu_sc as plsc

_NC = 2
_NS = 16
_CHUNK = 128
_RB = 400


@functools.lru_cache(maxsize=None)
def _make_spmm(n_pad, n_rows, width, rpt0_frac=0.9):
    blk = 8
    rpt_pair = n_rows // _NS
    rpt0 = int(round(rpt_pair * rpt0_frac / blk)) * blk
    rpt1 = rpt_pair - rpt0
    npt = n_pad // _NS

    mesh = plsc.VectorSubcoreMesh(core_axis_name="c", subcore_axis_name="s")

    @functools.partial(
        pl.kernel,
        out_type=jax.ShapeDtypeStruct((_NC, n_pad, width), jnp.float32),
        mesh=mesh,
        scratch_types=[
            pltpu.VMEM((blk, _CHUNK), jnp.int32),
            pltpu.VMEM((blk, _CHUNK), jnp.int32),
            pltpu.VMEM((_CHUNK, width), jnp.float32),
            pltpu.VMEM((_CHUNK, width), jnp.float32),
            pltpu.VMEM_SHARED((n_pad, width), jnp.float32),
            pltpu.SemaphoreType.DMA,
            pltpu.SemaphoreType.DMA,
        ],
    )
    def spmm(table, src2d, dst2d, out, src_v, dst_v, r0, r1, acc, g0, g1):
        rows = (r0, r1)
        gsem = (g0, g1)
        c = lax.axis_index("c")
        s = lax.axis_index("s")

        def zfill(i, carry):
            for g in range(width // 16):
                r0[i, pl.ds(g * 16, 16)] = jnp.zeros((16,), jnp.float32)
            return carry

        lax.fori_loop(0, _CHUNK, zfill, 0)
        nfull = npt // _CHUNK
        rem = npt % _CHUNK
        for k in range(nfull):
            pltpu.sync_copy(r0, acc.at[pl.ds(s * npt + k * _CHUNK, _CHUNK)])
        if rem:
            pltpu.sync_copy(r0.at[pl.ds(0, rem)],
                            acc.at[pl.ds(s * npt + nfull * _CHUNK, rem)])
        plsc.subcore_barrier()

        def run_share(base, n):
            def block(ib, carry):
                pltpu.sync_copy(src2d.at[pl.ds(base + ib * blk, blk)], src_v)
                pltpu.sync_copy(dst2d.at[pl.ds(base + ib * blk, blk)], dst_v)
                pltpu.async_copy(table.at[src_v.at[0]], rows[0], gsem[0])
                for j in range(blk):
                    b = j % 2
                    if j + 1 < blk:
                        pltpu.async_copy(table.at[src_v.at[j + 1]],
                                         rows[1 - b], gsem[1 - b])
                    pltpu.make_async_copy(table.at[src_v.at[j]], rows[b],
                                          gsem[b]).wait()
                    pltpu.sync_copy(rows[b], acc.at[dst_v.at[j]], add=True)
                return carry

            lax.fori_loop(0, n // blk, block, 0)

        @pl.when(c == 0)
        def _():
            run_share(s * rpt0, rpt0)

        @pl.when(c == 1)
        def _():
            run_share(_NS * rpt0 + s * rpt1, rpt1)

        plsc.subcore_barrier()
        for k in range(nfull):
            off = s * npt + k * _CHUNK
            pltpu.sync_copy(acc.at[pl.ds(off, _CHUNK)], r0)
            pltpu.sync_copy(r0, out.at[c, pl.ds(off, _CHUNK)])
        if rem:
            off = s * npt + nfull * _CHUNK
            pltpu.sync_copy(acc.at[pl.ds(off, rem)], r0.at[pl.ds(0, rem)])
            pltpu.sync_copy(r0.at[pl.ds(0, rem)], out.at[c, pl.ds(off, rem)])

    return spmm


@functools.lru_cache(maxsize=None)
def _make_deg(n_pad, n_rows):
    n_workers = _NC * _NS
    rpt = n_rows // n_workers

    mesh = plsc.VectorSubcoreMesh(core_axis_name="c", subcore_axis_name="s")

    @functools.partial(
        pl.kernel,
        out_type=jax.ShapeDtypeStruct((n_workers, n_pad), jnp.float32),
        mesh=mesh,
        compiler_params=pltpu.CompilerParams(needs_layout_passes=False),
        scratch_types=[
            pltpu.VMEM((rpt, _CHUNK), jnp.int32),
            pltpu.VMEM((n_pad,), jnp.float32),
        ],
    )
    def deg(dst2d, zeros, out, dst_v, acc):
        c = lax.axis_index("c")
        s = lax.axis_index("s")
        w = s * _NC + c
        pltpu.sync_copy(zeros, acc)
        pltpu.sync_copy(dst2d.at[pl.ds(w * rpt, rpt)], dst_v)

        def body(j, carry):
            ones_v = jnp.ones((16,), jnp.float32)
            for g in range(_CHUNK // 16):
                idx = dst_v[j, pl.ds(g * 16, 16)]
                plsc.addupdate_scatter(acc, [idx], ones_v)
            return carry

        lax.fori_loop(0, rpt, body, 0)
        pltpu.sync_copy(acc, out.at[w])

    return deg


def _tc_prep(degt_ref, x_ref, dinv_ref, xt_ref):
    deg = 1.0 + jnp.sum(degt_ref[...], axis=1, keepdims=True)
    dinv = lax.rsqrt(deg)
    dinv_ref[...] = dinv
    xt_ref[...] = x_ref[...] * dinv


def _tc_l1(p_ref, xt_ref, dinv_ref, W1_ref, b1_ref, W12_ref, h2t_ref):
    dinv = dinv_ref[...]
    ax = (p_ref[0] + p_ref[1] + xt_ref[...]) * dinv
    y1 = jnp.dot(ax, W1_ref[...], preferred_element_type=jnp.float32) + b1_ref[0:1, :]
    h2 = jnp.dot(y1, W12_ref[...], preferred_element_type=jnp.float32)
    h2t_ref[...] = h2 * dinv


def _tc_l2(q_ref, h2t_ref, dinv_ref, b12_ref, y2t_ref):
    dinv = dinv_ref[...]
    y2 = (q_ref[0] + q_ref[1] + h2t_ref[...]) * dinv + b12_ref[0:1, :]
    y2t_ref[...] = jnp.maximum(y2, 0.0) * dinv


def _tc_fin(r_ref, y2t_ref, dinv_ref, W2_ref, b2_ref, out_ref):
    ay2 = (r_ref[0] + r_ref[1] + y2t_ref[...]) * dinv_ref[...]
    out_ref[...] = (jnp.dot(ay2, W2_ref[...], preferred_element_type=jnp.float32)
                    + b2_ref[0:1, :])


def _row_spec(width):
    return pl.BlockSpec((_RB, width), lambda i: (i, 0))


def _pair_spec(width):
    return pl.BlockSpec((_NC, _RB, width), lambda i: (0, i, 0))


def _full_spec(shape):
    nd = len(shape)
    return pl.BlockSpec(shape, lambda i: (0,) * nd)


def kernel(x, edge_index, W1, b1, W12, b12, W2, b2):
    n_nodes, d_in = x.shape
    n_edges = edge_index.shape[1]
    d_mid = W1.shape[1]
    d_mid2 = W12.shape[1]
    d_out = W2.shape[1]
    grid = (n_nodes // _RB,)

    edge_align = _NC * _NS * 8 * _CHUNK
    e_pad = -(-n_edges // edge_align) * edge_align
    n_pad = -(-(n_nodes + 1) // (_NS * 8)) * (_NS * 8)
    n_rows = e_pad // _CHUNK

    ei = edge_index.astype(jnp.int32)
    pad_src = jnp.zeros((e_pad - n_edges,), jnp.int32)
    n_trash = n_pad - n_nodes
    pad_dst = n_nodes + jnp.arange(e_pad - n_edges, dtype=jnp.int32) % n_trash
    src2d = jnp.concatenate([ei[0], pad_src]).reshape(n_rows, _CHUNK)
    dst2d = jnp.concatenate([ei[1], pad_dst]).reshape(n_rows, _CHUNK)
    z1 = jnp.zeros((n_pad,), jnp.float32)
    b1r = jnp.broadcast_to(b1[None, :], (8, d_mid))
    b12r = jnp.broadcast_to(b12[None, :], (8, d_mid2))
    b2r = jnp.broadcast_to(b2[None, :], (8, d_out))

    degp = _make_deg(n_pad, n_rows)(dst2d, z1)
    degt = degp.T

    dinv, xt = pl.pallas_call(
        _tc_prep,
        grid=grid,
        in_specs=[pl.BlockSpec((_RB, _NC * _NS), lambda i: (i, 0)),
                  _row_spec(d_in)],
        out_specs=[_row_spec(1), _row_spec(d_in)],
        out_shape=[
            jax.ShapeDtypeStruct((n_nodes, 1), jnp.float32),
            jax.ShapeDtypeStruct((n_nodes, d_in), jnp.float32),
        ],
    )(degt, x)

    p = _make_spmm(n_pad, n_rows, d_in)(xt, src2d, dst2d)

    h2t = pl.pallas_call(
        _tc_l1,
        grid=grid,
        in_specs=[_pair_spec(d_in), _row_spec(d_in), _row_spec(1),
                  _full_spec(W1.shape), _full_spec((8, d_mid)),
                  _full_spec(W12.shape)],
        out_specs=_row_spec(d_mid2),
        out_shape=jax.ShapeDtypeStruct((n_nodes, d_mid2), jnp.float32),
    )(p, xt, dinv, W1, b1r, W12)

    q = _make_spmm(n_pad, n_rows, d_mid2)(h2t, src2d, dst2d)

    y2t = pl.pallas_call(
        _tc_l2,
        grid=grid,
        in_specs=[_pair_spec(d_mid2), _row_spec(d_mid2), _row_spec(1),
                  _full_spec((8, d_mid2))],
        out_specs=_row_spec(d_mid2),
        out_shape=jax.ShapeDtypeStruct((n_nodes, d_mid2), jnp.float32),
    )(q, h2t, dinv, b12r)

    r = _make_spmm(n_pad, n_rows, d_mid2)(y2t, src2d, dst2d)

    out = pl.pallas_call(
        _tc_fin,
        grid=grid,
        in_specs=[_pair_spec(d_mid2), _row_spec(d_mid2), _row_spec(1),
                  _full_spec(W2.shape), _full_spec((8, d_out))],
        out_specs=_row_spec(d_out),
        out_shape=jax.ShapeDtypeStruct((n_nodes, d_out), jnp.float32),
    )(r, y2t, dinv, W2, b2r)

    return out


# speedup vs baseline: 1.4138x; 1.0145x over previous
import functools

import jax
import jax.numpy as jnp
from jax import lax
from jax.experimental import pallas as pl
from jax.experimental.pallas import tpu as pltpu
from jax.experimental.pallas import tpu_sc as plsc

_NC = 2
_NS = 16
_CHUNK = 128
_RB = 400


@functools.lru_cache(maxsize=None)
def _make_spmm(n_pad, n_rows, width, rpt0_frac=0.95):
    blk = 8
    rpt_pair = n_rows // _NS
    rpt0 = int(round(rpt_pair * rpt0_frac / blk)) * blk
    rpt1 = rpt_pair - rpt0
    npt = n_pad // _NS

    mesh = plsc.VectorSubcoreMesh(core_axis_name="c", subcore_axis_name="s")

    @functools.partial(
        pl.kernel,
        out_type=jax.ShapeDtypeStruct((_NC, n_pad, width), jnp.float32),
        mesh=mesh,
        scratch_types=[
            pltpu.VMEM((blk, _CHUNK), jnp.int32),
            pltpu.VMEM((blk, _CHUNK), jnp.int32),
            pltpu.VMEM((_CHUNK, width), jnp.float32),
            pltpu.VMEM((_CHUNK, width), jnp.float32),
            pltpu.VMEM_SHARED((n_pad, width), jnp.float32),
            pltpu.SemaphoreType.DMA,
            pltpu.SemaphoreType.DMA,
        ],
    )
    def spmm(table, src2d, dst2d, out, src_v, dst_v, r0, r1, acc, g0, g1):
        rows = (r0, r1)
        gsem = (g0, g1)
        c = lax.axis_index("c")
        s = lax.axis_index("s")

        def zfill(i, carry):
            for g in range(width // 16):
                r0[i, pl.ds(g * 16, 16)] = jnp.zeros((16,), jnp.float32)
            return carry

        lax.fori_loop(0, _CHUNK, zfill, 0)
        nfull = npt // _CHUNK
        rem = npt % _CHUNK
        for k in range(nfull):
            pltpu.sync_copy(r0, acc.at[pl.ds(s * npt + k * _CHUNK, _CHUNK)])
        if rem:
            pltpu.sync_copy(r0.at[pl.ds(0, rem)],
                            acc.at[pl.ds(s * npt + nfull * _CHUNK, rem)])
        plsc.subcore_barrier()

        def run_share(base, n):
            def block(ib, carry):
                pltpu.sync_copy(src2d.at[pl.ds(base + ib * blk, blk)], src_v)
                pltpu.sync_copy(dst2d.at[pl.ds(base + ib * blk, blk)], dst_v)
                pltpu.async_copy(table.at[src_v.at[0]], rows[0], gsem[0])
                for j in range(blk):
                    b = j % 2
                    if j + 1 < blk:
                        pltpu.async_copy(table.at[src_v.at[j + 1]],
                                         rows[1 - b], gsem[1 - b])
                    pltpu.make_async_copy(table.at[src_v.at[j]], rows[b],
                                          gsem[b]).wait()
                    pltpu.sync_copy(rows[b], acc.at[dst_v.at[j]], add=True)
                return carry

            lax.fori_loop(0, n // blk, block, 0)

        @pl.when(c == 0)
        def _():
            run_share(s * rpt0, rpt0)

        @pl.when(c == 1)
        def _():
            run_share(_NS * rpt0 + s * rpt1, rpt1)

        plsc.subcore_barrier()
        for k in range(nfull):
            off = s * npt + k * _CHUNK
            pltpu.sync_copy(acc.at[pl.ds(off, _CHUNK)], r0)
            pltpu.sync_copy(r0, out.at[c, pl.ds(off, _CHUNK)])
        if rem:
            off = s * npt + nfull * _CHUNK
            pltpu.sync_copy(acc.at[pl.ds(off, rem)], r0.at[pl.ds(0, rem)])
            pltpu.sync_copy(r0.at[pl.ds(0, rem)], out.at[c, pl.ds(off, rem)])

    return spmm


@functools.lru_cache(maxsize=None)
def _make_deg(n_pad, n_rows):
    n_workers = _NC * _NS
    rpt = n_rows // n_workers

    mesh = plsc.VectorSubcoreMesh(core_axis_name="c", subcore_axis_name="s")

    @functools.partial(
        pl.kernel,
        out_type=jax.ShapeDtypeStruct((n_workers, n_pad), jnp.float32),
        mesh=mesh,
        compiler_params=pltpu.CompilerParams(needs_layout_passes=False),
        scratch_types=[
            pltpu.VMEM((rpt, _CHUNK), jnp.int32),
            pltpu.VMEM((n_pad,), jnp.float32),
        ],
    )
    def deg(dst2d, zeros, out, dst_v, acc):
        c = lax.axis_index("c")
        s = lax.axis_index("s")
        w = s * _NC + c
        pltpu.sync_copy(zeros, acc)
        pltpu.sync_copy(dst2d.at[pl.ds(w * rpt, rpt)], dst_v)

        def body(j, carry):
            ones_v = jnp.ones((16,), jnp.float32)
            for g in range(_CHUNK // 16):
                idx = dst_v[j, pl.ds(g * 16, 16)]
                plsc.addupdate_scatter(acc, [idx], ones_v)
            return carry

        lax.fori_loop(0, rpt, body, 0)
        pltpu.sync_copy(acc, out.at[w])

    return deg


def _tc_prep(degt_ref, x_ref, dinv_ref, xt_ref):
    deg = 1.0 + jnp.sum(degt_ref[...], axis=1, keepdims=True)
    dinv = lax.rsqrt(deg)
    dinv_ref[...] = dinv
    xt_ref[...] = x_ref[...] * dinv


def _tc_l1(p_ref, xt_ref, dinv_ref, W1_ref, b1_ref, W12_ref, h2t_ref):
    dinv = dinv_ref[...]
    ax = (p_ref[0] + p_ref[1] + xt_ref[...]) * dinv
    y1 = jnp.dot(ax, W1_ref[...], preferred_element_type=jnp.float32) + b1_ref[0:1, :]
    h2 = jnp.dot(y1, W12_ref[...], preferred_element_type=jnp.float32)
    h2t_ref[...] = h2 * dinv


def _tc_l2(q_ref, h2t_ref, dinv_ref, b12_ref, y2t_ref):
    dinv = dinv_ref[...]
    y2 = (q_ref[0] + q_ref[1] + h2t_ref[...]) * dinv + b12_ref[0:1, :]
    y2t_ref[...] = jnp.maximum(y2, 0.0) * dinv


def _tc_fin(r_ref, y2t_ref, dinv_ref, W2_ref, b2_ref, out_ref):
    ay2 = (r_ref[0] + r_ref[1] + y2t_ref[...]) * dinv_ref[...]
    out_ref[...] = (jnp.dot(ay2, W2_ref[...], preferred_element_type=jnp.float32)
                    + b2_ref[0:1, :])


def _row_spec(width):
    return pl.BlockSpec((_RB, width), lambda i: (i, 0))


def _pair_spec(width):
    return pl.BlockSpec((_NC, _RB, width), lambda i: (0, i, 0))


def _full_spec(shape):
    nd = len(shape)
    return pl.BlockSpec(shape, lambda i: (0,) * nd)


def kernel(x, edge_index, W1, b1, W12, b12, W2, b2):
    n_nodes, d_in = x.shape
    n_edges = edge_index.shape[1]
    d_mid = W1.shape[1]
    d_mid2 = W12.shape[1]
    d_out = W2.shape[1]
    grid = (n_nodes // _RB,)

    edge_align = _NC * _NS * 8 * _CHUNK
    e_pad = -(-n_edges // edge_align) * edge_align
    n_pad = -(-(n_nodes + 1) // (_NS * 8)) * (_NS * 8)
    n_rows = e_pad // _CHUNK

    ei = edge_index.astype(jnp.int32)
    pad_src = jnp.zeros((e_pad - n_edges,), jnp.int32)
    n_trash = n_pad - n_nodes
    pad_dst = n_nodes + jnp.arange(e_pad - n_edges, dtype=jnp.int32) % n_trash
    src2d = jnp.concatenate([ei[0], pad_src]).reshape(n_rows, _CHUNK)
    dst2d = jnp.concatenate([ei[1], pad_dst]).reshape(n_rows, _CHUNK)
    z1 = jnp.zeros((n_pad,), jnp.float32)
    b1r = jnp.broadcast_to(b1[None, :], (8, d_mid))
    b12r = jnp.broadcast_to(b12[None, :], (8, d_mid2))
    b2r = jnp.broadcast_to(b2[None, :], (8, d_out))

    degp = _make_deg(n_pad, n_rows)(dst2d, z1)
    degt = degp.T

    dinv, xt = pl.pallas_call(
        _tc_prep,
        grid=grid,
        in_specs=[pl.BlockSpec((_RB, _NC * _NS), lambda i: (i, 0)),
                  _row_spec(d_in)],
        out_specs=[_row_spec(1), _row_spec(d_in)],
        out_shape=[
            jax.ShapeDtypeStruct((n_nodes, 1), jnp.float32),
            jax.ShapeDtypeStruct((n_nodes, d_in), jnp.float32),
        ],
    )(degt, x)

    p = _make_spmm(n_pad, n_rows, d_in)(xt, src2d, dst2d)

    h2t = pl.pallas_call(
        _tc_l1,
        grid=grid,
        in_specs=[_pair_spec(d_in), _row_spec(d_in), _row_spec(1),
                  _full_spec(W1.shape), _full_spec((8, d_mid)),
                  _full_spec(W12.shape)],
        out_specs=_row_spec(d_mid2),
        out_shape=jax.ShapeDtypeStruct((n_nodes, d_mid2), jnp.float32),
    )(p, xt, dinv, W1, b1r, W12)

    q = _make_spmm(n_pad, n_rows, d_mid2)(h2t, src2d, dst2d)

    y2t = pl.pallas_call(
        _tc_l2,
        grid=grid,
        in_specs=[_pair_spec(d_mid2), _row_spec(d_mid2), _row_spec(1),
                  _full_spec((8, d_mid2))],
        out_specs=_row_spec(d_mid2),
        out_shape=jax.ShapeDtypeStruct((n_nodes, d_mid2), jnp.float32),
    )(q, h2t, dinv, b12r)

    r = _make_spmm(n_pad, n_rows, d_mid2)(y2t, src2d, dst2d)

    out = pl.pallas_call(
        _tc_fin,
        grid=grid,
        in_specs=[_pair_spec(d_mid2), _row_spec(d_mid2), _row_spec(1),
                  _full_spec(W2.shape), _full_spec((8, d_out))],
        out_specs=_row_spec(d_out),
        out_shape=jax.ShapeDtypeStruct((n_nodes, d_out), jnp.float32),
    )(r, y2t, dinv, W2, b2r)

    return out
